# baseline (device time: 16144 ns/iter reference)
import jax
import jax.numpy as jnp
from jax import lax
from jax.experimental import pallas as pl
from jax.experimental.pallas import tpu as pltpu

N_DEV = 4
E_PER = 2
N_SUB = 4


def kernel(x, router_W, route_idx, expert_W):
    n, d = x.shape
    hdim = expert_W.shape[-1]
    q = n // N_DEV
    sub = q // N_SUB

    def body(x_hbm, idx_ref, w_hbm, out_hbm,
             x_ref, w_ref, res, txbuf, rsbuf, in_sems, out_sems,
             rs_send, rs_recv, ag_send, ag_recv):
        my_pos = lax.axis_index("i")
        peers = [my_pos ^ 2, my_pos ^ 1, 3 - my_pos]

        cp_x = pltpu.make_async_copy(x_hbm, x_ref, in_sems.at[0])
        cp_w = pltpu.make_async_copy(w_hbm, w_ref, in_sems.at[1])
        cp_x.start()
        cp_w.start()

        barrier_sem = pltpu.get_barrier_semaphore()
        for nbr in peers:
            pl.semaphore_signal(
                barrier_sem, inc=1,
                device_id=(nbr,), device_id_type=pl.DeviceIdType.MESH,
            )
        cp_w.wait()
        cp_x.wait()

        wcat = w_ref[:, :, :].astype(jnp.bfloat16).reshape(E_PER * d, hdim)

        def quarter_partial(row_off):
            xq = x_ref[pl.ds(row_off, q), :].astype(jnp.bfloat16)
            iq = idx_ref[pl.ds(row_off, q), :]
            masked = [
                xq * (iq == my_pos * E_PER + k).astype(jnp.bfloat16)
                for k in range(E_PER)
            ]
            return jnp.dot(
                jnp.concatenate(masked, axis=1), wcat,
                preferred_element_type=jnp.float32,
            )

        def rs_rdma(j, s):
            slot = lax.rem(my_pos - j - 1 + N_DEV, N_DEV) * N_SUB + s
            return pltpu.make_async_remote_copy(
                src_ref=txbuf.at[pl.ds(j * q + s * sub, sub), :],
                dst_ref=rsbuf.at[slot],
                send_sem=rs_send.at[slot],
                recv_sem=rs_recv.at[slot],
                device_id=(j,), device_id_type=pl.DeviceIdType.MESH,
            )

        def ag_rdma(j, s):
            slot = lax.rem(my_pos - j - 1 + N_DEV, N_DEV) * N_SUB + s
            return pltpu.make_async_remote_copy(
                src_ref=txbuf.at[pl.ds(my_pos * q + s * sub, sub), :],
                dst_ref=txbuf.at[pl.ds(my_pos * q + s * sub, sub), :],
                send_sem=ag_send.at[slot],
                recv_sem=ag_recv.at[slot],
                device_id=(j,), device_id_type=pl.DeviceIdType.MESH,
            )

        def writeback(off, slot):
            cp = pltpu.make_async_copy(
                res.at[pl.ds(off, sub), :],
                out_hbm.at[pl.ds(off, sub), :],
                out_sems.at[slot],
            )
            cp.start()
            return cp

        rs_subs = [[] for _ in range(N_SUB)]
        for pi, j in enumerate(peers):
            txbuf[pl.ds(j * q, q), :] = quarter_partial(j * q).astype(jnp.bfloat16)
            if pi == 0:
                pl.semaphore_wait(barrier_sem, 3)
            r = rs_rdma(j, 0)
            r.start()
            rs_subs[0].append(r)
        for s in range(1, N_SUB):
            for j in peers:
                r = rs_rdma(j, s)
                r.start()
                rs_subs[s].append(r)
        res[pl.ds(my_pos * q, q), :] = quarter_partial(my_pos * q)

        ag_rdmas = []
        out_cps = []
        for s, rs_list in enumerate(rs_subs):
            for r in rs_list:
                r.wait_recv()
            off = my_pos * q + s * sub
            red = (
                res[pl.ds(off, sub), :]
                + rsbuf[0 * N_SUB + s].astype(jnp.float32)
                + rsbuf[1 * N_SUB + s].astype(jnp.float32)
                + rsbuf[2 * N_SUB + s].astype(jnp.float32)
            )
            res[pl.ds(off, sub), :] = red
            txbuf[pl.ds(off, sub), :] = red.astype(jnp.bfloat16)
            for j in peers:
                r = ag_rdma(j, s)
                r.start()
                t = lax.rem(my_pos - j - 1 + N_DEV, N_DEV)
                sender = lax.rem(my_pos + t + 1, N_DEV)
                ag_rdmas.append((r, sender, s))
            out_cps.append(writeback(off, s))

        for rs_list in rs_subs:
            for r in rs_list:
                r.wait_send()
        for i, (r, sender, s) in enumerate(ag_rdmas):
            r.wait_recv()
            off = sender * q + s * sub
            res[pl.ds(off, sub), :] = txbuf[pl.ds(off, sub), :].astype(jnp.float32)
            out_cps.append(writeback(off, N_SUB + i))
        for r, _, _ in ag_rdmas:
            r.wait_send()
        for cp in out_cps:
            cp.wait()

    return pl.pallas_call(
        body,
        out_shape=jax.ShapeDtypeStruct((n, hdim), jnp.float32),
        in_specs=[
            pl.BlockSpec(memory_space=pltpu.MemorySpace.HBM),
            pl.BlockSpec(memory_space=pltpu.VMEM),
            pl.BlockSpec(memory_space=pltpu.MemorySpace.HBM),
        ],
        out_specs=pl.BlockSpec(memory_space=pltpu.MemorySpace.HBM),
        scratch_shapes=[
            pltpu.VMEM((n, d), jnp.float32),
            pltpu.VMEM((E_PER, d, hdim), jnp.float32),
            pltpu.VMEM((n, hdim), jnp.float32),
            pltpu.VMEM((n, hdim), jnp.bfloat16),
            pltpu.VMEM((3 * N_SUB, sub, hdim), jnp.bfloat16),
            pltpu.SemaphoreType.DMA((2,)),
            pltpu.SemaphoreType.DMA((N_SUB + 3 * N_SUB,)),
            pltpu.SemaphoreType.DMA((3 * N_SUB,)),
            pltpu.SemaphoreType.DMA((3 * N_SUB,)),
            pltpu.SemaphoreType.DMA((3 * N_SUB,)),
            pltpu.SemaphoreType.DMA((3 * N_SUB,)),
        ],
        compiler_params=pltpu.CompilerParams(collective_id=0),
    )(x, route_idx, expert_W)


# device time: 15650 ns/iter; 1.0316x vs baseline; 1.0316x over previous
import jax
import jax.numpy as jnp
from jax import lax
from jax.experimental import pallas as pl
from jax.experimental.pallas import tpu as pltpu

N_DEV = 4
E_PER = 2
N_SUB = 4


def kernel(x, router_W, route_idx, expert_W):
    n, d = x.shape
    hdim = expert_W.shape[-1]
    q = n // N_DEV
    sub = q // N_SUB

    def body(x_ref, idx_ref, w_ref, out_ref,
             txbuf, rsbuf, rs_send, rs_recv, ag_send, ag_recv):
        my_pos = lax.axis_index("i")
        peers = [my_pos ^ 2, my_pos ^ 1, 3 - my_pos]

        barrier_sem = pltpu.get_barrier_semaphore()
        for nbr in peers:
            pl.semaphore_signal(
                barrier_sem, inc=1,
                device_id=(nbr,), device_id_type=pl.DeviceIdType.MESH,
            )

        wcat = w_ref[:, :, :].astype(jnp.bfloat16).reshape(E_PER * d, hdim)

        def quarter_partial(row_off):
            xq = x_ref[pl.ds(row_off, q), :].astype(jnp.bfloat16)
            iq = idx_ref[pl.ds(row_off, q), :]
            masked = [
                xq * (iq == my_pos * E_PER + k).astype(jnp.bfloat16)
                for k in range(E_PER)
            ]
            return jnp.dot(
                jnp.concatenate(masked, axis=1), wcat,
                preferred_element_type=jnp.float32,
            )

        def rs_rdma(j, s):
            slot = lax.rem(my_pos - j - 1 + N_DEV, N_DEV) * N_SUB + s
            return pltpu.make_async_remote_copy(
                src_ref=txbuf.at[pl.ds(j * q + s * sub, sub), :],
                dst_ref=rsbuf.at[slot],
                send_sem=rs_send.at[slot],
                recv_sem=rs_recv.at[slot],
                device_id=(j,), device_id_type=pl.DeviceIdType.MESH,
            )

        def ag_rdma(j, s):
            slot = lax.rem(my_pos - j - 1 + N_DEV, N_DEV) * N_SUB + s
            return pltpu.make_async_remote_copy(
                src_ref=txbuf.at[pl.ds(my_pos * q + s * sub, sub), :],
                dst_ref=txbuf.at[pl.ds(my_pos * q + s * sub, sub), :],
                send_sem=ag_send.at[slot],
                recv_sem=ag_recv.at[slot],
                device_id=(j,), device_id_type=pl.DeviceIdType.MESH,
            )

        rs_subs = [[] for _ in range(N_SUB)]
        for pi, j in enumerate(peers):
            txbuf[pl.ds(j * q, q), :] = quarter_partial(j * q).astype(jnp.bfloat16)
            if pi == 0:
                pl.semaphore_wait(barrier_sem, 3)
            r = rs_rdma(j, 0)
            r.start()
            rs_subs[0].append(r)
        for s in range(1, N_SUB):
            for j in peers:
                r = rs_rdma(j, s)
                r.start()
                rs_subs[s].append(r)
        own = quarter_partial(my_pos * q)
        out_ref[pl.ds(my_pos * q, q), :] = own

        ag_rdmas = []
        for s, rs_list in enumerate(rs_subs):
            for r in rs_list:
                r.wait_recv()
            off = my_pos * q + s * sub
            red = (
                out_ref[pl.ds(off, sub), :]
                + rsbuf[0 * N_SUB + s].astype(jnp.float32)
                + rsbuf[1 * N_SUB + s].astype(jnp.float32)
                + rsbuf[2 * N_SUB + s].astype(jnp.float32)
            )
            out_ref[pl.ds(off, sub), :] = red
            txbuf[pl.ds(off, sub), :] = red.astype(jnp.bfloat16)
            for j in peers:
                r = ag_rdma(j, s)
                r.start()
                t = lax.rem(my_pos - j - 1 + N_DEV, N_DEV)
                sender = lax.rem(my_pos + t + 1, N_DEV)
                ag_rdmas.append((r, sender, s))

        for rs_list in rs_subs:
            for r in rs_list:
                r.wait_send()
        for r, sender, s in ag_rdmas:
            r.wait_recv()
            off = sender * q + s * sub
            out_ref[pl.ds(off, sub), :] = (
                txbuf[pl.ds(off, sub), :].astype(jnp.float32)
            )
        for r, _, _ in ag_rdmas:
            r.wait_send()

    return pl.pallas_call(
        body,
        out_shape=jax.ShapeDtypeStruct((n, hdim), jnp.float32),
        in_specs=[pl.BlockSpec(memory_space=pltpu.VMEM)] * 3,
        out_specs=pl.BlockSpec(memory_space=pltpu.VMEM),
        scratch_shapes=[
            pltpu.VMEM((n, hdim), jnp.bfloat16),
            pltpu.VMEM((3 * N_SUB, sub, hdim), jnp.bfloat16),
            pltpu.SemaphoreType.DMA((3 * N_SUB,)),
            pltpu.SemaphoreType.DMA((3 * N_SUB,)),
            pltpu.SemaphoreType.DMA((3 * N_SUB,)),
            pltpu.SemaphoreType.DMA((3 * N_SUB,)),
        ],
        compiler_params=pltpu.CompilerParams(collective_id=0),
    )(x, route_idx, expert_W)


# device time: 15422 ns/iter; 1.0468x vs baseline; 1.0148x over previous
import jax
import jax.numpy as jnp
from jax import lax
from jax.experimental import pallas as pl
from jax.experimental.pallas import tpu as pltpu

N_DEV = 4
E_PER = 2
N_SUB = 4


def kernel(x, router_W, route_idx, expert_W):
    n, d = x.shape
    hdim = expert_W.shape[-1]
    q = n // N_DEV
    sub = q // N_SUB

    def body(x_ref, idx_ref, w_ref, out_ref,
             ownbuf, txbuf, rsbuf, rs_send, rs_recv, ag_send, ag_recv):
        my_pos = lax.axis_index("i")
        peers = [my_pos ^ 2, my_pos ^ 1, 3 - my_pos]

        barrier_sem = pltpu.get_barrier_semaphore()
        for nbr in peers:
            pl.semaphore_signal(
                barrier_sem, inc=1,
                device_id=(nbr,), device_id_type=pl.DeviceIdType.MESH,
            )

        wcat = w_ref[:, :, :].astype(jnp.bfloat16).reshape(E_PER * d, hdim)

        def quarter_partial(row_off):
            xq = x_ref[pl.ds(row_off, q), :].astype(jnp.bfloat16)
            iq = idx_ref[pl.ds(row_off, q), :]
            masked = [
                xq * (iq == my_pos * E_PER + k).astype(jnp.bfloat16)
                for k in range(E_PER)
            ]
            return jnp.dot(
                jnp.concatenate(masked, axis=1), wcat,
                preferred_element_type=jnp.float32,
            )

        def rs_rdma(j, s):
            slot = lax.rem(my_pos - j - 1 + N_DEV, N_DEV) * N_SUB + s
            return pltpu.make_async_remote_copy(
                src_ref=txbuf.at[pl.ds(j * q + s * sub, sub), :],
                dst_ref=rsbuf.at[slot],
                send_sem=rs_send.at[slot],
                recv_sem=rs_recv.at[slot],
                device_id=(j,), device_id_type=pl.DeviceIdType.MESH,
            )

        def ag_rdma(j, s):
            slot = lax.rem(my_pos - j - 1 + N_DEV, N_DEV) * N_SUB + s
            return pltpu.make_async_remote_copy(
                src_ref=out_ref.at[pl.ds(my_pos * q + s * sub, sub), :],
                dst_ref=out_ref.at[pl.ds(my_pos * q + s * sub, sub), :],
                send_sem=ag_send.at[slot],
                recv_sem=ag_recv.at[slot],
                device_id=(j,), device_id_type=pl.DeviceIdType.MESH,
            )

        rs_subs = [[] for _ in range(N_SUB)]
        for pi, j in enumerate(peers):
            txbuf[pl.ds(j * q, q), :] = quarter_partial(j * q).astype(jnp.bfloat16)
            if pi == 0:
                pl.semaphore_wait(barrier_sem, 3)
            r = rs_rdma(j, 0)
            r.start()
            rs_subs[0].append(r)
        for s in range(1, N_SUB):
            for j in peers:
                r = rs_rdma(j, s)
                r.start()
                rs_subs[s].append(r)
        ownbuf[:, :] = quarter_partial(my_pos * q)

        ag_rdmas = []
        for s, rs_list in enumerate(rs_subs):
            for r in rs_list:
                r.wait_recv()
            red = (
                ownbuf[pl.ds(s * sub, sub), :]
                + rsbuf[0 * N_SUB + s].astype(jnp.float32)
                + rsbuf[1 * N_SUB + s].astype(jnp.float32)
                + rsbuf[2 * N_SUB + s].astype(jnp.float32)
            )
            out_ref[pl.ds(my_pos * q + s * sub, sub), :] = red.astype(jnp.bfloat16)
            for j in peers:
                r = ag_rdma(j, s)
                r.start()
                ag_rdmas.append(r)

        for rs_list in rs_subs:
            for r in rs_list:
                r.wait_send()
        for r in ag_rdmas:
            r.wait_recv()
        for r in ag_rdmas:
            r.wait_send()

    return pl.pallas_call(
        body,
        out_shape=jax.ShapeDtypeStruct((n, hdim), jnp.bfloat16),
        in_specs=[pl.BlockSpec(memory_space=pltpu.VMEM)] * 3,
        out_specs=pl.BlockSpec(memory_space=pltpu.VMEM),
        scratch_shapes=[
            pltpu.VMEM((q, hdim), jnp.float32),
            pltpu.VMEM((n, hdim), jnp.bfloat16),
            pltpu.VMEM((3 * N_SUB, sub, hdim), jnp.bfloat16),
            pltpu.SemaphoreType.DMA((3 * N_SUB,)),
            pltpu.SemaphoreType.DMA((3 * N_SUB,)),
            pltpu.SemaphoreType.DMA((3 * N_SUB,)),
            pltpu.SemaphoreType.DMA((3 * N_SUB,)),
        ],
        compiler_params=pltpu.CompilerParams(collective_id=0),
    )(x, route_idx, expert_W)


# device time: 15154 ns/iter; 1.0653x vs baseline; 1.0177x over previous
import jax
import jax.numpy as jnp
from jax import lax
from jax.experimental import pallas as pl
from jax.experimental.pallas import tpu as pltpu

N_DEV = 4
E_PER = 2
N_SUB = 4


def kernel(x, router_W, route_idx, expert_W):
    n, d = x.shape
    hdim = expert_W.shape[-1]
    q = n // N_DEV
    sub = q // N_SUB

    def body(x_ref, idx_ref, w_ref, out_ref,
             ownbuf, txbuf, rsbuf, rs_send, rs_recv, ag_send, ag_recv):
        my_pos = lax.axis_index("i")
        peers = [my_pos ^ 2, my_pos ^ 1, 3 - my_pos]

        barrier_sem = pltpu.get_barrier_semaphore()
        for nbr in peers:
            pl.semaphore_signal(
                barrier_sem, inc=1,
                device_id=(nbr,), device_id_type=pl.DeviceIdType.MESH,
            )

        wcat = w_ref[:, :, :].astype(jnp.bfloat16).reshape(E_PER * d, hdim)

        def quarter_partial(row_off):
            xq = x_ref[pl.ds(row_off, q), :].astype(jnp.bfloat16)
            iq = idx_ref[pl.ds(row_off, q), :]
            masked = [
                xq * (iq == my_pos * E_PER + k).astype(jnp.bfloat16)
                for k in range(E_PER)
            ]
            return jnp.dot(
                jnp.concatenate(masked, axis=1), wcat,
                preferred_element_type=jnp.float32,
            )

        def rs_rdma(j, s):
            slot = lax.rem(my_pos - j - 1 + N_DEV, N_DEV) * N_SUB + s
            return pltpu.make_async_remote_copy(
                src_ref=txbuf.at[pl.ds(j * q + s * sub, sub), :],
                dst_ref=rsbuf.at[slot],
                send_sem=rs_send.at[slot],
                recv_sem=rs_recv.at[slot],
                device_id=(j,), device_id_type=pl.DeviceIdType.MESH,
            )

        def ag_rdma(j, s):
            slot = lax.rem(my_pos - j - 1 + N_DEV, N_DEV) * N_SUB + s
            return pltpu.make_async_remote_copy(
                src_ref=out_ref.at[pl.ds(my_pos * q + s * sub, sub), :],
                dst_ref=out_ref.at[pl.ds(my_pos * q + s * sub, sub), :],
                send_sem=ag_send.at[slot],
                recv_sem=ag_recv.at[slot],
                device_id=(j,), device_id_type=pl.DeviceIdType.MESH,
            )

        for j in peers:
            txbuf[pl.ds(j * q, q), :] = quarter_partial(j * q).astype(jnp.bfloat16)
        ownbuf[:, :] = quarter_partial(my_pos * q)
        pl.semaphore_wait(barrier_sem, 3)
        rs_subs = [[] for _ in range(N_SUB)]
        for s in range(N_SUB):
            for j in peers:
                r = rs_rdma(j, s)
                r.start()
                rs_subs[s].append(r)

        ag_rdmas = []
        for s, rs_list in enumerate(rs_subs):
            for r in rs_list:
                r.wait_recv()
            red = (
                ownbuf[pl.ds(s * sub, sub), :]
                + rsbuf[0 * N_SUB + s].astype(jnp.float32)
                + rsbuf[1 * N_SUB + s].astype(jnp.float32)
                + rsbuf[2 * N_SUB + s].astype(jnp.float32)
            )
            out_ref[pl.ds(my_pos * q + s * sub, sub), :] = red.astype(jnp.bfloat16)
            for j in peers:
                r = ag_rdma(j, s)
                r.start()
                ag_rdmas.append(r)

        for rs_list in rs_subs:
            for r in rs_list:
                r.wait_send()
        for r in ag_rdmas:
            r.wait_recv()
        for r in ag_rdmas:
            r.wait_send()

    return pl.pallas_call(
        body,
        out_shape=jax.ShapeDtypeStruct((n, hdim), jnp.bfloat16),
        in_specs=[pl.BlockSpec(memory_space=pltpu.VMEM)] * 3,
        out_specs=pl.BlockSpec(memory_space=pltpu.VMEM),
        scratch_shapes=[
            pltpu.VMEM((q, hdim), jnp.float32),
            pltpu.VMEM((n, hdim), jnp.bfloat16),
            pltpu.VMEM((3 * N_SUB, sub, hdim), jnp.bfloat16),
            pltpu.SemaphoreType.DMA((3 * N_SUB,)),
            pltpu.SemaphoreType.DMA((3 * N_SUB,)),
            pltpu.SemaphoreType.DMA((3 * N_SUB,)),
            pltpu.SemaphoreType.DMA((3 * N_SUB,)),
        ],
        compiler_params=pltpu.CompilerParams(collective_id=0),
    )(x, route_idx, expert_W)
